# baseline (speedup 1.0000x reference)
import functools

import jax
import jax.numpy as jnp
from jax import lax
from jax.experimental import pallas as pl
from jax.experimental.pallas import tpu as pltpu
from jax.experimental.pallas import tpu_sc as plsc

N = 10000
NP = 10240
E = 160000
EP = 163840
NW = 32
EPW = EP // NW
CHUNK = 4096
NCH = EP // CHUNK
GPC = CHUNK // 16
IN_DIM = 256
H1 = 512
H2 = 300
H2P = 320
OUT_DIM = 40


@functools.lru_cache(maxsize=None)
def _mesh():
    return plsc.VectorSubcoreMesh(
        core_axis_name="c", subcore_axis_name="s", num_cores=2, num_subcores=16
    )


def _worker_id():
    return lax.axis_index("s") * 2 + lax.axis_index("c")


@functools.lru_cache(maxsize=None)
def _build_deg():
    @functools.partial(
        pl.kernel,
        out_type=jax.ShapeDtypeStruct((2 * NP,), jnp.float32),
        mesh=_mesh(),
        compiler_params=pltpu.CompilerParams(needs_layout_passes=False),
        scratch_types=(
            pltpu.VMEM((NP,), jnp.float32),
            pltpu.VMEM((EPW,), jnp.int32),
            pltpu.VMEM((EPW,), jnp.float32),
            pltpu.VMEM_SHARED((16 * NP,), jnp.float32),
            pltpu.VMEM((16 * 640,), jnp.float32),
            pltpu.VMEM((640,), jnp.float32),
        ),
    )
    def deg_kernel(rc, ew, deg2, degbuf, rcbuf, ewbuf, shared, red, accb):
        cid = lax.axis_index("c")
        sid = lax.axis_index("s")
        wid = sid * 2 + cid
        z16 = jnp.zeros((16,), jnp.float32)

        @pl.loop(0, NP // 16, unroll=8)
        def _(i):
            degbuf[pl.ds(i * 16, 16)] = z16

        base = wid * EPW
        pltpu.sync_copy(rc.at[pl.ds(base, EPW)], rcbuf)
        pltpu.sync_copy(ew.at[pl.ds(base, EPW)], ewbuf)

        @plsc.parallel_loop(0, EPW // 16, unroll=4)
        def _(g):
            rcv = rcbuf[pl.ds(g * 16, 16)]
            rowi = rcv & 0xFFFF
            plsc.addupdate_scatter(degbuf, [rowi], ewbuf[pl.ds(g * 16, 16)])

        pltpu.sync_copy(degbuf, shared.at[pl.ds(sid * NP, NP)])
        plsc.subcore_barrier()
        for j in range(16):
            pltpu.sync_copy(shared.at[pl.ds(j * NP + sid * 640, 640)],
                            red.at[pl.ds(j * 640, 640)])

        @pl.loop(0, 40)
        def _(i):
            acc = red[pl.ds(i * 16, 16)]
            for j in range(1, 16):
                acc = acc + red[pl.ds(j * 640 + i * 16, 16)]
            accb[pl.ds(i * 16, 16)] = acc

        pltpu.sync_copy(accb, deg2.at[pl.ds(cid * NP + sid * 640, 640)])

    return deg_kernel


@functools.lru_cache(maxsize=None)
def _build_lapw():
    @functools.partial(
        pl.kernel,
        out_type=jax.ShapeDtypeStruct((EP,), jnp.float32),
        mesh=_mesh(),
        compiler_params=pltpu.CompilerParams(needs_layout_passes=False),
        scratch_types=(
            pltpu.VMEM((NP,), jnp.float32),
            pltpu.VMEM((NP,), jnp.float32),
            pltpu.VMEM((EPW,), jnp.int32),
            pltpu.VMEM((EPW,), jnp.float32),
            pltpu.VMEM((EPW,), jnp.float32),
        ),
    )
    def lapw_kernel(deg2, rc, ew, lapw, disbuf, tbuf, rcbuf, ewbuf, lwbuf):
        wid = _worker_id()
        pltpu.sync_copy(deg2.at[pl.ds(0, NP)], disbuf)
        pltpu.sync_copy(deg2.at[pl.ds(NP, NP)], tbuf)

        @pl.loop(0, NP // 16, unroll=2)
        def _(i):
            d = disbuf[pl.ds(i * 16, 16)] + tbuf[pl.ds(i * 16, 16)]
            ii = lax.bitcast_convert_type(d, jnp.int32)
            ii = 0x5F3759DF - lax.shift_right_arithmetic(ii, 1)
            y = lax.bitcast_convert_type(ii, jnp.float32)
            for _unused in range(4):
                y = y * (1.5 - 0.5 * d * y * y)
            disbuf[pl.ds(i * 16, 16)] = jnp.where(d > 0.0, y, 0.0)

        base = wid * EPW
        pltpu.sync_copy(rc.at[pl.ds(base, EPW)], rcbuf)
        pltpu.sync_copy(ew.at[pl.ds(base, EPW)], ewbuf)

        @plsc.parallel_loop(0, EPW // 16, unroll=4)
        def _(g):
            rcv = rcbuf[pl.ds(g * 16, 16)]
            rowi = rcv & 0xFFFF
            coli = lax.shift_right_arithmetic(rcv, 16)
            a = plsc.load_gather(disbuf, [rowi])
            b = plsc.load_gather(disbuf, [coli])
            lwbuf[pl.ds(g * 16, 16)] = -(a * ewbuf[pl.ds(g * 16, 16)] * b)

        pltpu.sync_copy(lwbuf, lapw.at[pl.ds(base, EPW)])

    return lapw_kernel


@functools.lru_cache(maxsize=None)
def _build_prop(D):
    FPW = D // NW
    C = 4
    assert FPW % C == 0
    SCANS = FPW // C

    scratch = (
        [pltpu.VMEM((NP,), jnp.float32) for _ in range(2 * C)]
        + [pltpu.VMEM((CHUNK,), jnp.int32) for _ in range(2)]
        + [pltpu.VMEM((CHUNK,), jnp.float32) for _ in range(2)]
        + [pltpu.SemaphoreType.DMA, pltpu.SemaphoreType.DMA]
    )

    @functools.partial(
        pl.kernel,
        out_type=jax.ShapeDtypeStruct((D * NP,), jnp.float32),
        mesh=_mesh(),
        compiler_params=pltpu.CompilerParams(needs_layout_passes=False),
        scratch_types=tuple(scratch),
    )
    def prop(zT, rc, w, outT, *sc):
        zb = sc[0:C]
        ob = sc[C:2 * C]
        rcb = sc[2 * C:2 * C + 2]
        wb = sc[2 * C + 2:2 * C + 4]
        sems = sc[2 * C + 4:2 * C + 6]
        wid = _worker_id()
        z16 = jnp.zeros((16,), jnp.float32)

        def start(chunk_idx, p):
            off = chunk_idx * CHUNK
            pltpu.async_copy(rc.at[pl.ds(off, CHUNK)], rcb[p], sems[p])
            pltpu.async_copy(w.at[pl.ds(off, CHUNK)], wb[p], sems[p])

        def wait(p):
            pltpu.make_async_copy(rc.at[pl.ds(0, CHUNK)], rcb[p], sems[p]).wait()
            pltpu.make_async_copy(w.at[pl.ds(0, CHUNK)], wb[p], sems[p]).wait()

        def process(p):
            @plsc.parallel_loop(0, GPC, unroll=6)
            def _(g):
                rcv = rcb[p][pl.ds(g * 16, 16)]
                rowi = rcv & 0xFFFF
                coli = lax.shift_right_arithmetic(rcv, 16)
                wv = wb[p][pl.ds(g * 16, 16)]
                vals = [plsc.load_gather(zb[q], [rowi]) for q in range(C)]
                for q in range(C):
                    plsc.addupdate_scatter(ob[q], [coli], vals[q] * wv)

        for scan in range(SCANS):
            for q in range(C):
                f = wid * FPW + scan * C + q
                pltpu.sync_copy(zT.at[pl.ds(f * NP, NP)], zb[q])

            @pl.loop(0, NP // 16, unroll=8)
            def _(i):
                for q in range(C):
                    ob[q][pl.ds(i * 16, 16)] = z16

            start(0, 0)

            @pl.loop(0, NCH // 2)
            def _(h):
                ch = h * 2
                start(ch + 1, 1)
                wait(0)
                process(0)

                @pl.when(ch + 2 < NCH)
                def _():
                    start(ch + 2, 0)

                wait(1)
                process(1)

            for q in range(C):
                f = wid * FPW + scan * C + q
                pltpu.sync_copy(ob[q], outT.at[pl.ds(f * NP, NP)])

    return prop


def _prop(D, zT, rc, w):
    return _build_prop(D)(zT.reshape(D * NP), rc, w).reshape(D, NP)


_BN = 1024


def _mm3(w0, w1, w2, z0, z1, z2):
    M, D = w0.shape

    def body(w0r, w1r, w2r, z0r, z1r, z2r, outr):
        acc = jnp.dot(w0r[...], z0r[...], preferred_element_type=jnp.float32)
        acc += jnp.dot(w1r[...], z1r[...], preferred_element_type=jnp.float32)
        acc += jnp.dot(w2r[...], z2r[...], preferred_element_type=jnp.float32)
        outr[...] = acc

    wspec = pl.BlockSpec((M, D), lambda j: (0, 0))
    zspec = pl.BlockSpec((D, _BN), lambda j: (0, j))
    return pl.pallas_call(
        body,
        grid=(NP // _BN,),
        in_specs=[wspec, wspec, wspec, zspec, zspec, zspec],
        out_specs=pl.BlockSpec((M, _BN), lambda j: (0, j)),
        out_shape=jax.ShapeDtypeStruct((M, NP), jnp.float32),
    )(w0, w1, w2, z0, z1, z2)


def _bn_stats(h, g_t, be_t):
    M = h.shape[0]

    def body(hr, gr, br, sr, tr, acc1, acc2):
        j = pl.program_id(0)

        @pl.when(j == 0)
        def _():
            acc1[...] = jnp.zeros_like(acc1)
            acc2[...] = jnp.zeros_like(acc2)

        b = hr[...]
        acc1[...] += b.reshape(M, _BN // 128, 128).sum(axis=1)
        acc2[...] += (b * b).reshape(M, _BN // 128, 128).sum(axis=1)
        ssum = acc1[...].sum(axis=1, keepdims=True)
        sq = acc2[...].sum(axis=1, keepdims=True)
        mean = ssum / N
        var = sq / N - mean * mean
        rstd = lax.rsqrt(var + 1e-5)
        s = gr[...][:, :1] * rstd
        t = br[...][:, :1] - mean * s
        sr[...] = jnp.broadcast_to(s, (M, 128))
        tr[...] = jnp.broadcast_to(t, (M, 128))

    sspec = pl.BlockSpec((M, 128), lambda j: (0, 0))
    return pl.pallas_call(
        body,
        grid=(NP // _BN,),
        in_specs=[pl.BlockSpec((M, _BN), lambda j: (0, j)), sspec, sspec],
        out_specs=[sspec, sspec],
        out_shape=[jax.ShapeDtypeStruct((M, 128), jnp.float32)] * 2,
        scratch_shapes=[pltpu.VMEM((M, 128), jnp.float32)] * 2,
    )(h, g_t, be_t)


def _bn_apply(h, s_t, t_t):
    M = h.shape[0]

    def body(hr, sr, tr, outr):
        j = pl.program_id(0)
        col = j * _BN + lax.broadcasted_iota(jnp.int32, (M, _BN), 1)
        y = hr[...] * sr[...][:, :1] + tr[...][:, :1]
        outr[...] = jnp.where(col < N, y, 0.0)

    sspec = pl.BlockSpec((M, 128), lambda j: (0, 0))
    bspec = pl.BlockSpec((M, _BN), lambda j: (0, j))
    return pl.pallas_call(
        body,
        grid=(NP // _BN,),
        in_specs=[bspec, sspec, sspec],
        out_specs=bspec,
        out_shape=jax.ShapeDtypeStruct((M, NP), jnp.float32),
    )(h, s_t, t_t)


def _final(wlt, h, bl_t):
    def body(wr, hr, br, outr):
        z = jnp.dot(wr[...], hr[...], preferred_element_type=jnp.float32)
        z += br[...][:, :1]
        m = jnp.max(z, axis=0, keepdims=True)
        zc = z - m
        lse = jnp.log(jnp.sum(jnp.exp(zc), axis=0, keepdims=True))
        outr[...] = (zc - lse).T

    return pl.pallas_call(
        body,
        grid=(NP // _BN,),
        in_specs=[
            pl.BlockSpec((OUT_DIM, H2P), lambda j: (0, 0)),
            pl.BlockSpec((H2P, _BN), lambda j: (0, j)),
            pl.BlockSpec((OUT_DIM, 128), lambda j: (0, 0)),
        ],
        out_specs=pl.BlockSpec((_BN, OUT_DIM), lambda j: (j, 0)),
        out_shape=jax.ShapeDtypeStruct((NP, OUT_DIM), jnp.float32),
    )(wlt, h, bl_t)


def kernel(x, edge_index, edge_weight, W1, b1, g1, be1, W2, b2, g2, be2, Wl, bl):
    row = edge_index[0]
    col = edge_index[1]
    rc_raw = jnp.bitwise_or(row, col << 16)
    rc_raw_p = jnp.concatenate([rc_raw, jnp.zeros((EP - E,), jnp.int32)])
    ew_raw_p = jnp.concatenate([edge_weight, jnp.zeros((EP - E,), jnp.float32)])
    deg2 = _build_deg()(rc_raw_p, ew_raw_p)
    lapw_raw = _build_lapw()(deg2, rc_raw_p, ew_raw_p)

    key = ((col & 15) << 4) | ((row - col) & 15)
    perm = jnp.argsort(key, stable=True).reshape(16, E // 16).T.reshape(-1)
    rc_p = jnp.concatenate([rc_raw[perm], jnp.zeros((EP - E,), jnp.int32)])
    lapw = jnp.concatenate([lapw_raw[:E][perm],
                            jnp.zeros((EP - E,), jnp.float32)])

    xT = jnp.pad(x, ((0, NP - N), (0, 0))).T

    T1 = _prop(IN_DIM, xT, rc_p, lapw)
    P2 = _prop(IN_DIM, T1, rc_p, lapw)

    A1 = (W1[0] - W1[2]).T
    B1 = W1[1].T
    C1 = 2.0 * W1[2].T
    h1 = _mm3(A1, B1, C1, xT, T1, P2)

    g1t = jnp.tile(g1[:, None], (1, 128))
    be1t = jnp.tile(be1[:, None], (1, 128))
    s1, t1 = _bn_stats(h1, g1t, be1t)
    h1bn = _bn_apply(h1, s1, t1)

    U1 = _prop(H1, h1bn, rc_p, lapw)
    U2 = _prop(H1, U1, rc_p, lapw)

    pad2 = ((0, H2P - H2), (0, 0))
    A2 = jnp.pad((W2[0] - W2[2]).T, pad2)
    B2 = jnp.pad(W2[1].T, pad2)
    C2 = jnp.pad(2.0 * W2[2].T, pad2)
    h2 = _mm3(A2, B2, C2, h1bn, U1, U2)

    g2t = jnp.tile(jnp.pad(g2, (0, H2P - H2))[:, None], (1, 128))
    be2t = jnp.tile(jnp.pad(be2, (0, H2P - H2))[:, None], (1, 128))
    s2, t2 = _bn_stats(h2, g2t, be2t)
    h2bn = _bn_apply(h2, s2, t2)

    wlt = jnp.pad(Wl.T, ((0, 0), (0, H2P - H2)))
    bl_t = jnp.tile(bl[:, None], (1, 128))
    out = _final(wlt, h2bn, bl_t)
    return out[:N]


# speedup vs baseline: 1.0120x; 1.0120x over previous
import functools

import jax
import jax.numpy as jnp
from jax import lax
from jax.experimental import pallas as pl
from jax.experimental.pallas import tpu as pltpu
from jax.experimental.pallas import tpu_sc as plsc

N = 10000
NP = 10240
E = 160000
EP = 163840
NW = 32
EPW = EP // NW
CHUNK = 4096
NCH = EP // CHUNK
GPC = CHUNK // 16
IN_DIM = 256
H1 = 512
H2 = 300
H2P = 320
OUT_DIM = 40


@functools.lru_cache(maxsize=None)
def _mesh():
    return plsc.VectorSubcoreMesh(
        core_axis_name="c", subcore_axis_name="s", num_cores=2, num_subcores=16
    )


def _worker_id():
    return lax.axis_index("s") * 2 + lax.axis_index("c")


@functools.lru_cache(maxsize=None)
def _build_deg():
    @functools.partial(
        pl.kernel,
        out_type=jax.ShapeDtypeStruct((2 * NP,), jnp.float32),
        mesh=_mesh(),
        compiler_params=pltpu.CompilerParams(needs_layout_passes=False),
        scratch_types=(
            pltpu.VMEM((NP,), jnp.float32),
            pltpu.VMEM((EPW,), jnp.int32),
            pltpu.VMEM((EPW,), jnp.float32),
            pltpu.VMEM_SHARED((16 * NP,), jnp.float32),
            pltpu.VMEM((16 * 640,), jnp.float32),
            pltpu.VMEM((640,), jnp.float32),
        ),
    )
    def deg_kernel(rc, ew, deg2, degbuf, rcbuf, ewbuf, shared, red, accb):
        cid = lax.axis_index("c")
        sid = lax.axis_index("s")
        wid = sid * 2 + cid
        z16 = jnp.zeros((16,), jnp.float32)

        @pl.loop(0, NP // 16, unroll=8)
        def _(i):
            degbuf[pl.ds(i * 16, 16)] = z16

        base = wid * EPW
        pltpu.sync_copy(rc.at[pl.ds(base, EPW)], rcbuf)
        pltpu.sync_copy(ew.at[pl.ds(base, EPW)], ewbuf)

        @plsc.parallel_loop(0, EPW // 16, unroll=4)
        def _(g):
            rcv = rcbuf[pl.ds(g * 16, 16)]
            rowi = rcv & 0xFFFF
            plsc.addupdate_scatter(degbuf, [rowi], ewbuf[pl.ds(g * 16, 16)])

        pltpu.sync_copy(degbuf, shared.at[pl.ds(sid * NP, NP)])
        plsc.subcore_barrier()
        for j in range(16):
            pltpu.sync_copy(shared.at[pl.ds(j * NP + sid * 640, 640)],
                            red.at[pl.ds(j * 640, 640)])

        @pl.loop(0, 40)
        def _(i):
            acc = red[pl.ds(i * 16, 16)]
            for j in range(1, 16):
                acc = acc + red[pl.ds(j * 640 + i * 16, 16)]
            accb[pl.ds(i * 16, 16)] = acc

        pltpu.sync_copy(accb, deg2.at[pl.ds(cid * NP + sid * 640, 640)])

    return deg_kernel


@functools.lru_cache(maxsize=None)
def _build_lapw():
    @functools.partial(
        pl.kernel,
        out_type=jax.ShapeDtypeStruct((EP,), jnp.float32),
        mesh=_mesh(),
        compiler_params=pltpu.CompilerParams(needs_layout_passes=False),
        scratch_types=(
            pltpu.VMEM((NP,), jnp.float32),
            pltpu.VMEM((NP,), jnp.float32),
            pltpu.VMEM((EPW,), jnp.int32),
            pltpu.VMEM((EPW,), jnp.float32),
            pltpu.VMEM((EPW,), jnp.float32),
        ),
    )
    def lapw_kernel(deg2, rc, ew, lapw, disbuf, tbuf, rcbuf, ewbuf, lwbuf):
        wid = _worker_id()
        pltpu.sync_copy(deg2.at[pl.ds(0, NP)], disbuf)
        pltpu.sync_copy(deg2.at[pl.ds(NP, NP)], tbuf)

        @pl.loop(0, NP // 16, unroll=2)
        def _(i):
            d = disbuf[pl.ds(i * 16, 16)] + tbuf[pl.ds(i * 16, 16)]
            ii = lax.bitcast_convert_type(d, jnp.int32)
            ii = 0x5F3759DF - lax.shift_right_arithmetic(ii, 1)
            y = lax.bitcast_convert_type(ii, jnp.float32)
            for _unused in range(4):
                y = y * (1.5 - 0.5 * d * y * y)
            disbuf[pl.ds(i * 16, 16)] = jnp.where(d > 0.0, y, 0.0)

        base = wid * EPW
        pltpu.sync_copy(rc.at[pl.ds(base, EPW)], rcbuf)
        pltpu.sync_copy(ew.at[pl.ds(base, EPW)], ewbuf)

        @plsc.parallel_loop(0, EPW // 16, unroll=4)
        def _(g):
            rcv = rcbuf[pl.ds(g * 16, 16)]
            rowi = rcv & 0xFFFF
            coli = lax.shift_right_arithmetic(rcv, 16)
            a = plsc.load_gather(disbuf, [rowi])
            b = plsc.load_gather(disbuf, [coli])
            lwbuf[pl.ds(g * 16, 16)] = -(a * ewbuf[pl.ds(g * 16, 16)] * b)

        pltpu.sync_copy(lwbuf, lapw.at[pl.ds(base, EPW)])

    return lapw_kernel


@functools.lru_cache(maxsize=None)
def _build_prop(D):
    FPW = D // NW
    C = 4
    assert FPW % C == 0
    SCANS = FPW // C

    scratch = (
        [pltpu.VMEM((NP,), jnp.float32) for _ in range(2 * C)]
        + [pltpu.VMEM((CHUNK,), jnp.int32) for _ in range(2)]
        + [pltpu.VMEM((CHUNK,), jnp.float32) for _ in range(2)]
        + [pltpu.SemaphoreType.DMA, pltpu.SemaphoreType.DMA]
    )

    @functools.partial(
        pl.kernel,
        out_type=jax.ShapeDtypeStruct((D * NP,), jnp.float32),
        mesh=_mesh(),
        compiler_params=pltpu.CompilerParams(needs_layout_passes=False),
        scratch_types=tuple(scratch),
    )
    def prop(zT, rc, w, outT, *sc):
        zb = sc[0:C]
        ob = sc[C:2 * C]
        rcb = sc[2 * C:2 * C + 2]
        wb = sc[2 * C + 2:2 * C + 4]
        sems = sc[2 * C + 4:2 * C + 6]
        wid = _worker_id()
        z16 = jnp.zeros((16,), jnp.float32)

        def start(chunk_idx, p):
            off = chunk_idx * CHUNK
            pltpu.async_copy(rc.at[pl.ds(off, CHUNK)], rcb[p], sems[p])
            pltpu.async_copy(w.at[pl.ds(off, CHUNK)], wb[p], sems[p])

        def wait(p):
            pltpu.make_async_copy(rc.at[pl.ds(0, CHUNK)], rcb[p], sems[p]).wait()
            pltpu.make_async_copy(w.at[pl.ds(0, CHUNK)], wb[p], sems[p]).wait()

        def process(p):
            @plsc.parallel_loop(0, GPC, unroll=4)
            def _(g):
                rcv = rcb[p][pl.ds(g * 16, 16)]
                rowi = rcv & 0xFFFF
                coli = lax.shift_right_arithmetic(rcv, 16)
                wv = wb[p][pl.ds(g * 16, 16)]
                vals = [plsc.load_gather(zb[q], [rowi]) for q in range(C)]
                for q in range(C):
                    plsc.addupdate_scatter(ob[q], [coli], vals[q] * wv)

        for scan in range(SCANS):
            for q in range(C):
                f = wid * FPW + scan * C + q
                pltpu.sync_copy(zT.at[pl.ds(f * NP, NP)], zb[q])

            @pl.loop(0, NP // 16, unroll=8)
            def _(i):
                for q in range(C):
                    ob[q][pl.ds(i * 16, 16)] = z16

            start(0, 0)

            @pl.loop(0, NCH // 2)
            def _(h):
                ch = h * 2
                start(ch + 1, 1)
                wait(0)
                process(0)

                @pl.when(ch + 2 < NCH)
                def _():
                    start(ch + 2, 0)

                wait(1)
                process(1)

            for q in range(C):
                f = wid * FPW + scan * C + q
                pltpu.sync_copy(ob[q], outT.at[pl.ds(f * NP, NP)])

    return prop


def _prop(D, zT, rc, w):
    return _build_prop(D)(zT.reshape(D * NP), rc, w).reshape(D, NP)


_BN = 1024


def _mm3(w0, w1, w2, z0, z1, z2):
    M, D = w0.shape

    def body(w0r, w1r, w2r, z0r, z1r, z2r, outr):
        acc = jnp.dot(w0r[...], z0r[...], preferred_element_type=jnp.float32)
        acc += jnp.dot(w1r[...], z1r[...], preferred_element_type=jnp.float32)
        acc += jnp.dot(w2r[...], z2r[...], preferred_element_type=jnp.float32)
        outr[...] = acc

    wspec = pl.BlockSpec((M, D), lambda j: (0, 0))
    zspec = pl.BlockSpec((D, _BN), lambda j: (0, j))
    return pl.pallas_call(
        body,
        grid=(NP // _BN,),
        in_specs=[wspec, wspec, wspec, zspec, zspec, zspec],
        out_specs=pl.BlockSpec((M, _BN), lambda j: (0, j)),
        out_shape=jax.ShapeDtypeStruct((M, NP), jnp.float32),
    )(w0, w1, w2, z0, z1, z2)


def _bn_stats(h, g_t, be_t):
    M = h.shape[0]

    def body(hr, gr, br, sr, tr, acc1, acc2):
        j = pl.program_id(0)

        @pl.when(j == 0)
        def _():
            acc1[...] = jnp.zeros_like(acc1)
            acc2[...] = jnp.zeros_like(acc2)

        b = hr[...]
        acc1[...] += b.reshape(M, _BN // 128, 128).sum(axis=1)
        acc2[...] += (b * b).reshape(M, _BN // 128, 128).sum(axis=1)
        ssum = acc1[...].sum(axis=1, keepdims=True)
        sq = acc2[...].sum(axis=1, keepdims=True)
        mean = ssum / N
        var = sq / N - mean * mean
        rstd = lax.rsqrt(var + 1e-5)
        s = gr[...][:, :1] * rstd
        t = br[...][:, :1] - mean * s
        sr[...] = jnp.broadcast_to(s, (M, 128))
        tr[...] = jnp.broadcast_to(t, (M, 128))

    sspec = pl.BlockSpec((M, 128), lambda j: (0, 0))
    return pl.pallas_call(
        body,
        grid=(NP // _BN,),
        in_specs=[pl.BlockSpec((M, _BN), lambda j: (0, j)), sspec, sspec],
        out_specs=[sspec, sspec],
        out_shape=[jax.ShapeDtypeStruct((M, 128), jnp.float32)] * 2,
        scratch_shapes=[pltpu.VMEM((M, 128), jnp.float32)] * 2,
    )(h, g_t, be_t)


def _bn_apply(h, s_t, t_t):
    M = h.shape[0]

    def body(hr, sr, tr, outr):
        j = pl.program_id(0)
        col = j * _BN + lax.broadcasted_iota(jnp.int32, (M, _BN), 1)
        y = hr[...] * sr[...][:, :1] + tr[...][:, :1]
        outr[...] = jnp.where(col < N, y, 0.0)

    sspec = pl.BlockSpec((M, 128), lambda j: (0, 0))
    bspec = pl.BlockSpec((M, _BN), lambda j: (0, j))
    return pl.pallas_call(
        body,
        grid=(NP // _BN,),
        in_specs=[bspec, sspec, sspec],
        out_specs=bspec,
        out_shape=jax.ShapeDtypeStruct((M, NP), jnp.float32),
    )(h, s_t, t_t)


def _final(wlt, h, bl_t):
    def body(wr, hr, br, outr):
        z = jnp.dot(wr[...], hr[...], preferred_element_type=jnp.float32)
        z += br[...][:, :1]
        m = jnp.max(z, axis=0, keepdims=True)
        zc = z - m
        lse = jnp.log(jnp.sum(jnp.exp(zc), axis=0, keepdims=True))
        outr[...] = (zc - lse).T

    return pl.pallas_call(
        body,
        grid=(NP // _BN,),
        in_specs=[
            pl.BlockSpec((OUT_DIM, H2P), lambda j: (0, 0)),
            pl.BlockSpec((H2P, _BN), lambda j: (0, j)),
            pl.BlockSpec((OUT_DIM, 128), lambda j: (0, 0)),
        ],
        out_specs=pl.BlockSpec((_BN, OUT_DIM), lambda j: (j, 0)),
        out_shape=jax.ShapeDtypeStruct((NP, OUT_DIM), jnp.float32),
    )(wlt, h, bl_t)


def kernel(x, edge_index, edge_weight, W1, b1, g1, be1, W2, b2, g2, be2, Wl, bl):
    row = edge_index[0]
    col = edge_index[1]
    rc_raw = jnp.bitwise_or(row, col << 16)
    rc_raw_p = jnp.concatenate([rc_raw, jnp.zeros((EP - E,), jnp.int32)])
    ew_raw_p = jnp.concatenate([edge_weight, jnp.zeros((EP - E,), jnp.float32)])
    deg2 = _build_deg()(rc_raw_p, ew_raw_p)
    lapw_raw = _build_lapw()(deg2, rc_raw_p, ew_raw_p)

    key = ((col & 15) << 4) | ((row - col) & 15)
    perm = jnp.argsort(key, stable=True).reshape(16, E // 16).T.reshape(-1)
    rc_p = jnp.concatenate([rc_raw[perm], jnp.zeros((EP - E,), jnp.int32)])
    lapw = jnp.concatenate([lapw_raw[:E][perm],
                            jnp.zeros((EP - E,), jnp.float32)])

    xT = jnp.pad(x, ((0, NP - N), (0, 0))).T

    T1 = _prop(IN_DIM, xT, rc_p, lapw)
    P2 = _prop(IN_DIM, T1, rc_p, lapw)

    A1 = (W1[0] - W1[2]).T
    B1 = W1[1].T
    C1 = 2.0 * W1[2].T
    h1 = _mm3(A1, B1, C1, xT, T1, P2)

    g1t = jnp.tile(g1[:, None], (1, 128))
    be1t = jnp.tile(be1[:, None], (1, 128))
    s1, t1 = _bn_stats(h1, g1t, be1t)
    h1bn = _bn_apply(h1, s1, t1)

    U1 = _prop(H1, h1bn, rc_p, lapw)
    U2 = _prop(H1, U1, rc_p, lapw)

    pad2 = ((0, H2P - H2), (0, 0))
    A2 = jnp.pad((W2[0] - W2[2]).T, pad2)
    B2 = jnp.pad(W2[1].T, pad2)
    C2 = jnp.pad(2.0 * W2[2].T, pad2)
    h2 = _mm3(A2, B2, C2, h1bn, U1, U2)

    g2t = jnp.tile(jnp.pad(g2, (0, H2P - H2))[:, None], (1, 128))
    be2t = jnp.tile(jnp.pad(be2, (0, H2P - H2))[:, None], (1, 128))
    s2, t2 = _bn_stats(h2, g2t, be2t)

    wlt0 = jnp.pad(Wl.T, ((0, 0), (0, H2P - H2)))
    wlt = wlt0 * s2[:, 0][None, :]
    blf = wlt0 @ t2[:, 0] + bl
    bl_t = jnp.tile(blf[:, None], (1, 128))
    out = _final(wlt, h2, bl_t)
    return out[:N]


# speedup vs baseline: 1.0220x; 1.0099x over previous
import functools

import jax
import jax.numpy as jnp
from jax import lax
from jax.experimental import pallas as pl
from jax.experimental.pallas import tpu as pltpu
from jax.experimental.pallas import tpu_sc as plsc

N = 10000
NP = 10240
E = 160000
EP = 163840
NW = 32
EPW = EP // NW
CHUNK = 4096
NCH = EP // CHUNK
GPC = CHUNK // 16
IN_DIM = 256
H1 = 512
H2 = 300
H2P = 320
OUT_DIM = 40


@functools.lru_cache(maxsize=None)
def _mesh():
    return plsc.VectorSubcoreMesh(
        core_axis_name="c", subcore_axis_name="s", num_cores=2, num_subcores=16
    )


def _worker_id():
    return lax.axis_index("s") * 2 + lax.axis_index("c")


@functools.lru_cache(maxsize=None)
def _build_deg():
    @functools.partial(
        pl.kernel,
        out_type=jax.ShapeDtypeStruct((2 * NP,), jnp.float32),
        mesh=_mesh(),
        compiler_params=pltpu.CompilerParams(needs_layout_passes=False),
        scratch_types=(
            pltpu.VMEM((NP,), jnp.float32),
            pltpu.VMEM((EPW,), jnp.int32),
            pltpu.VMEM((EPW,), jnp.float32),
            pltpu.VMEM_SHARED((16 * NP,), jnp.float32),
            pltpu.VMEM((16 * 640,), jnp.float32),
            pltpu.VMEM((640,), jnp.float32),
        ),
    )
    def deg_kernel(rc, ew, deg2, degbuf, rcbuf, ewbuf, shared, red, accb):
        cid = lax.axis_index("c")
        sid = lax.axis_index("s")
        wid = sid * 2 + cid
        z16 = jnp.zeros((16,), jnp.float32)

        @pl.loop(0, NP // 16, unroll=8)
        def _(i):
            degbuf[pl.ds(i * 16, 16)] = z16

        base = wid * EPW
        pltpu.sync_copy(rc.at[pl.ds(base, EPW)], rcbuf)
        pltpu.sync_copy(ew.at[pl.ds(base, EPW)], ewbuf)

        @plsc.parallel_loop(0, EPW // 16, unroll=4)
        def _(g):
            rcv = rcbuf[pl.ds(g * 16, 16)]
            rowi = rcv & 0xFFFF
            plsc.addupdate_scatter(degbuf, [rowi], ewbuf[pl.ds(g * 16, 16)])

        pltpu.sync_copy(degbuf, shared.at[pl.ds(sid * NP, NP)])
        plsc.subcore_barrier()
        for j in range(16):
            pltpu.sync_copy(shared.at[pl.ds(j * NP + sid * 640, 640)],
                            red.at[pl.ds(j * 640, 640)])

        @pl.loop(0, 40)
        def _(i):
            acc = red[pl.ds(i * 16, 16)]
            for j in range(1, 16):
                acc = acc + red[pl.ds(j * 640 + i * 16, 16)]
            accb[pl.ds(i * 16, 16)] = acc

        pltpu.sync_copy(accb, deg2.at[pl.ds(cid * NP + sid * 640, 640)])

    return deg_kernel


@functools.lru_cache(maxsize=None)
def _build_lapw():
    @functools.partial(
        pl.kernel,
        out_type=jax.ShapeDtypeStruct((EP,), jnp.float32),
        mesh=_mesh(),
        compiler_params=pltpu.CompilerParams(needs_layout_passes=False),
        scratch_types=(
            pltpu.VMEM((NP,), jnp.float32),
            pltpu.VMEM((NP,), jnp.float32),
            pltpu.VMEM((EPW,), jnp.int32),
            pltpu.VMEM((EPW,), jnp.float32),
            pltpu.VMEM((EPW,), jnp.float32),
        ),
    )
    def lapw_kernel(deg2, rc, ew, lapw, disbuf, tbuf, rcbuf, ewbuf, lwbuf):
        wid = _worker_id()
        pltpu.sync_copy(deg2.at[pl.ds(0, NP)], disbuf)
        pltpu.sync_copy(deg2.at[pl.ds(NP, NP)], tbuf)

        @pl.loop(0, NP // 16, unroll=2)
        def _(i):
            d = disbuf[pl.ds(i * 16, 16)] + tbuf[pl.ds(i * 16, 16)]
            ii = lax.bitcast_convert_type(d, jnp.int32)
            ii = 0x5F3759DF - lax.shift_right_arithmetic(ii, 1)
            y = lax.bitcast_convert_type(ii, jnp.float32)
            for _unused in range(4):
                y = y * (1.5 - 0.5 * d * y * y)
            disbuf[pl.ds(i * 16, 16)] = jnp.where(d > 0.0, y, 0.0)

        base = wid * EPW
        pltpu.sync_copy(rc.at[pl.ds(base, EPW)], rcbuf)
        pltpu.sync_copy(ew.at[pl.ds(base, EPW)], ewbuf)

        @plsc.parallel_loop(0, EPW // 16, unroll=4)
        def _(g):
            rcv = rcbuf[pl.ds(g * 16, 16)]
            rowi = rcv & 0xFFFF
            coli = lax.shift_right_arithmetic(rcv, 16)
            a = plsc.load_gather(disbuf, [rowi])
            b = plsc.load_gather(disbuf, [coli])
            lwbuf[pl.ds(g * 16, 16)] = -(a * ewbuf[pl.ds(g * 16, 16)] * b)

        pltpu.sync_copy(lwbuf, lapw.at[pl.ds(base, EPW)])

    return lapw_kernel


@functools.lru_cache(maxsize=None)
def _build_prop(D):
    FPW = D // NW
    C = 4
    assert FPW % C == 0
    SCANS = FPW // C

    scratch = (
        [pltpu.VMEM((NP,), jnp.float32) for _ in range(2 * C)]
        + [pltpu.VMEM((CHUNK,), jnp.int32) for _ in range(2)]
        + [pltpu.VMEM((CHUNK,), jnp.float32) for _ in range(2)]
        + [pltpu.SemaphoreType.DMA, pltpu.SemaphoreType.DMA]
    )

    @functools.partial(
        pl.kernel,
        out_type=jax.ShapeDtypeStruct((D * NP,), jnp.float32),
        mesh=_mesh(),
        compiler_params=pltpu.CompilerParams(needs_layout_passes=False),
        scratch_types=tuple(scratch),
    )
    def prop(zT, rc, w, outT, *sc):
        zb = sc[0:C]
        ob = sc[C:2 * C]
        rcb = sc[2 * C:2 * C + 2]
        wb = sc[2 * C + 2:2 * C + 4]
        sems = sc[2 * C + 4:2 * C + 6]
        wid = _worker_id()
        z16 = jnp.zeros((16,), jnp.float32)

        def start(chunk_idx, p):
            off = chunk_idx * CHUNK
            pltpu.async_copy(rc.at[pl.ds(off, CHUNK)], rcb[p], sems[p])
            pltpu.async_copy(w.at[pl.ds(off, CHUNK)], wb[p], sems[p])

        def wait(p):
            pltpu.make_async_copy(rc.at[pl.ds(0, CHUNK)], rcb[p], sems[p]).wait()
            pltpu.make_async_copy(w.at[pl.ds(0, CHUNK)], wb[p], sems[p]).wait()

        def process(p):
            @plsc.parallel_loop(0, GPC, unroll=4)
            def _(g):
                rcv = rcb[p][pl.ds(g * 16, 16)]
                rowi = rcv & 0xFFFF
                coli = lax.shift_right_arithmetic(rcv, 16)
                wv = wb[p][pl.ds(g * 16, 16)]
                vals = [plsc.load_gather(zb[q], [rowi]) for q in range(C)]
                for q in range(C):
                    plsc.addupdate_scatter(ob[q], [coli], vals[q] * wv)

        for scan in range(SCANS):
            for q in range(C):
                f = wid * FPW + scan * C + q
                pltpu.sync_copy(zT.at[pl.ds(f * NP, NP)], zb[q])

            @pl.loop(0, NP // 16, unroll=8)
            def _(i):
                for q in range(C):
                    ob[q][pl.ds(i * 16, 16)] = z16

            start(0, 0)

            @pl.loop(0, NCH // 2)
            def _(h):
                ch = h * 2
                start(ch + 1, 1)
                wait(0)
                process(0)

                @pl.when(ch + 2 < NCH)
                def _():
                    start(ch + 2, 0)

                wait(1)
                process(1)

            for q in range(C):
                f = wid * FPW + scan * C + q
                pltpu.sync_copy(ob[q], outT.at[pl.ds(f * NP, NP)])

    return prop


def _prop(D, zT, rc, w):
    return _build_prop(D)(zT.reshape(D * NP), rc, w).reshape(D, NP)


_BN = 2048


def _mm3(w0, w1, w2, z0, z1, z2):
    M, D = w0.shape

    def body(w0r, w1r, w2r, z0r, z1r, z2r, outr):
        acc = jnp.dot(w0r[...], z0r[...], preferred_element_type=jnp.float32)
        acc += jnp.dot(w1r[...], z1r[...], preferred_element_type=jnp.float32)
        acc += jnp.dot(w2r[...], z2r[...], preferred_element_type=jnp.float32)
        outr[...] = acc

    wspec = pl.BlockSpec((M, D), lambda j: (0, 0))
    zspec = pl.BlockSpec((D, _BN), lambda j: (0, j))
    return pl.pallas_call(
        body,
        grid=(NP // _BN,),
        in_specs=[wspec, wspec, wspec, zspec, zspec, zspec],
        out_specs=pl.BlockSpec((M, _BN), lambda j: (0, j)),
        out_shape=jax.ShapeDtypeStruct((M, NP), jnp.float32),
    )(w0, w1, w2, z0, z1, z2)


def _bn_stats(h, g_t, be_t):
    M = h.shape[0]

    def body(hr, gr, br, sr, tr, acc1, acc2):
        j = pl.program_id(0)

        @pl.when(j == 0)
        def _():
            acc1[...] = jnp.zeros_like(acc1)
            acc2[...] = jnp.zeros_like(acc2)

        b = hr[...]
        acc1[...] += b.reshape(M, _BN // 128, 128).sum(axis=1)
        acc2[...] += (b * b).reshape(M, _BN // 128, 128).sum(axis=1)
        ssum = acc1[...].sum(axis=1, keepdims=True)
        sq = acc2[...].sum(axis=1, keepdims=True)
        mean = ssum / N
        var = sq / N - mean * mean
        rstd = lax.rsqrt(var + 1e-5)
        s = gr[...][:, :1] * rstd
        t = br[...][:, :1] - mean * s
        sr[...] = jnp.broadcast_to(s, (M, 128))
        tr[...] = jnp.broadcast_to(t, (M, 128))

    sspec = pl.BlockSpec((M, 128), lambda j: (0, 0))
    return pl.pallas_call(
        body,
        grid=(NP // _BN,),
        in_specs=[pl.BlockSpec((M, _BN), lambda j: (0, j)), sspec, sspec],
        out_specs=[sspec, sspec],
        out_shape=[jax.ShapeDtypeStruct((M, 128), jnp.float32)] * 2,
        scratch_shapes=[pltpu.VMEM((M, 128), jnp.float32)] * 2,
    )(h, g_t, be_t)


def _bn_apply(h, s_t, t_t):
    M = h.shape[0]

    def body(hr, sr, tr, outr):
        j = pl.program_id(0)
        col = j * _BN + lax.broadcasted_iota(jnp.int32, (M, _BN), 1)
        y = hr[...] * sr[...][:, :1] + tr[...][:, :1]
        outr[...] = jnp.where(col < N, y, 0.0)

    sspec = pl.BlockSpec((M, 128), lambda j: (0, 0))
    bspec = pl.BlockSpec((M, _BN), lambda j: (0, j))
    return pl.pallas_call(
        body,
        grid=(NP // _BN,),
        in_specs=[bspec, sspec, sspec],
        out_specs=bspec,
        out_shape=jax.ShapeDtypeStruct((M, NP), jnp.float32),
    )(h, s_t, t_t)


def _final(wlt, h, bl_t):
    def body(wr, hr, br, outr):
        z = jnp.dot(wr[...], hr[...], preferred_element_type=jnp.float32)
        z += br[...][:, :1]
        m = jnp.max(z, axis=0, keepdims=True)
        zc = z - m
        lse = jnp.log(jnp.sum(jnp.exp(zc), axis=0, keepdims=True))
        outr[...] = (zc - lse).T

    return pl.pallas_call(
        body,
        grid=(NP // _BN,),
        in_specs=[
            pl.BlockSpec((OUT_DIM, H2P), lambda j: (0, 0)),
            pl.BlockSpec((H2P, _BN), lambda j: (0, j)),
            pl.BlockSpec((OUT_DIM, 128), lambda j: (0, 0)),
        ],
        out_specs=pl.BlockSpec((_BN, OUT_DIM), lambda j: (j, 0)),
        out_shape=jax.ShapeDtypeStruct((NP, OUT_DIM), jnp.float32),
    )(wlt, h, bl_t)


def kernel(x, edge_index, edge_weight, W1, b1, g1, be1, W2, b2, g2, be2, Wl, bl):
    row = edge_index[0]
    col = edge_index[1]
    rc_raw = jnp.bitwise_or(row, col << 16)
    rc_raw_p = jnp.concatenate([rc_raw, jnp.zeros((EP - E,), jnp.int32)])
    ew_raw_p = jnp.concatenate([edge_weight, jnp.zeros((EP - E,), jnp.float32)])
    deg2 = _build_deg()(rc_raw_p, ew_raw_p)
    lapw_raw = _build_lapw()(deg2, rc_raw_p, ew_raw_p)

    key = ((col & 15) << 4) | ((row - col) & 15)
    perm = jnp.argsort(key, stable=True).reshape(16, E // 16).T.reshape(-1)
    rc_p = jnp.concatenate([rc_raw[perm], jnp.zeros((EP - E,), jnp.int32)])
    lapw = jnp.concatenate([lapw_raw[:E][perm],
                            jnp.zeros((EP - E,), jnp.float32)])

    xT = jnp.pad(x, ((0, NP - N), (0, 0))).T

    T1 = _prop(IN_DIM, xT, rc_p, lapw)
    P2 = _prop(IN_DIM, T1, rc_p, lapw)

    A1 = (W1[0] - W1[2]).T
    B1 = W1[1].T
    C1 = 2.0 * W1[2].T
    h1 = _mm3(A1, B1, C1, xT, T1, P2)

    g1t = jnp.tile(g1[:, None], (1, 128))
    be1t = jnp.tile(be1[:, None], (1, 128))
    s1, t1 = _bn_stats(h1, g1t, be1t)
    h1bn = _bn_apply(h1, s1, t1)

    U1 = _prop(H1, h1bn, rc_p, lapw)
    U2 = _prop(H1, U1, rc_p, lapw)

    pad2 = ((0, H2P - H2), (0, 0))
    A2 = jnp.pad((W2[0] - W2[2]).T, pad2)
    B2 = jnp.pad(W2[1].T, pad2)
    C2 = jnp.pad(2.0 * W2[2].T, pad2)
    h2 = _mm3(A2, B2, C2, h1bn, U1, U2)

    g2t = jnp.tile(jnp.pad(g2, (0, H2P - H2))[:, None], (1, 128))
    be2t = jnp.tile(jnp.pad(be2, (0, H2P - H2))[:, None], (1, 128))
    s2, t2 = _bn_stats(h2, g2t, be2t)

    wlt0 = jnp.pad(Wl.T, ((0, 0), (0, H2P - H2)))
    wlt = wlt0 * s2[:, 0][None, :]
    blf = wlt0 @ t2[:, 0] + bl
    bl_t = jnp.tile(blf[:, None], (1, 128))
    out = _final(wlt, h2, bl_t)
    return out[:N]


# speedup vs baseline: 1.0225x; 1.0005x over previous
import functools

import jax
import jax.numpy as jnp
from jax import lax
from jax.experimental import pallas as pl
from jax.experimental.pallas import tpu as pltpu
from jax.experimental.pallas import tpu_sc as plsc

N = 10000
NP = 10240
E = 160000
EP = 163840
NW = 32
EPW = EP // NW
CHUNK = 4096
NCH = EP // CHUNK
GPC = CHUNK // 16
IN_DIM = 256
H1 = 512
H2 = 300
H2P = 320
OUT_DIM = 40


@functools.lru_cache(maxsize=None)
def _mesh():
    return plsc.VectorSubcoreMesh(
        core_axis_name="c", subcore_axis_name="s", num_cores=2, num_subcores=16
    )


def _worker_id():
    return lax.axis_index("s") * 2 + lax.axis_index("c")


@functools.lru_cache(maxsize=None)
def _build_deg():
    @functools.partial(
        pl.kernel,
        out_type=jax.ShapeDtypeStruct((2 * NP,), jnp.float32),
        mesh=_mesh(),
        compiler_params=pltpu.CompilerParams(needs_layout_passes=False),
        scratch_types=(
            pltpu.VMEM((NP,), jnp.float32),
            pltpu.VMEM((EPW,), jnp.int32),
            pltpu.VMEM((EPW,), jnp.float32),
            pltpu.VMEM_SHARED((16 * NP,), jnp.float32),
            pltpu.VMEM((16 * 640,), jnp.float32),
            pltpu.VMEM((640,), jnp.float32),
        ),
    )
    def deg_kernel(rc, ew, deg2, degbuf, rcbuf, ewbuf, shared, red, accb):
        cid = lax.axis_index("c")
        sid = lax.axis_index("s")
        wid = sid * 2 + cid
        z16 = jnp.zeros((16,), jnp.float32)

        @pl.loop(0, NP // 16, unroll=8)
        def _(i):
            degbuf[pl.ds(i * 16, 16)] = z16

        base = wid * EPW
        pltpu.sync_copy(rc.at[pl.ds(base, EPW)], rcbuf)
        pltpu.sync_copy(ew.at[pl.ds(base, EPW)], ewbuf)

        @plsc.parallel_loop(0, EPW // 16, unroll=4)
        def _(g):
            rcv = rcbuf[pl.ds(g * 16, 16)]
            rowi = rcv & 0xFFFF
            plsc.addupdate_scatter(degbuf, [rowi], ewbuf[pl.ds(g * 16, 16)])

        pltpu.sync_copy(degbuf, shared.at[pl.ds(sid * NP, NP)])
        plsc.subcore_barrier()
        for j in range(16):
            pltpu.sync_copy(shared.at[pl.ds(j * NP + sid * 640, 640)],
                            red.at[pl.ds(j * 640, 640)])

        @pl.loop(0, 40)
        def _(i):
            acc = red[pl.ds(i * 16, 16)]
            for j in range(1, 16):
                acc = acc + red[pl.ds(j * 640 + i * 16, 16)]
            accb[pl.ds(i * 16, 16)] = acc

        pltpu.sync_copy(accb, deg2.at[pl.ds(cid * NP + sid * 640, 640)])

    return deg_kernel


@functools.lru_cache(maxsize=None)
def _build_lapw():
    @functools.partial(
        pl.kernel,
        out_type=jax.ShapeDtypeStruct((EP,), jnp.float32),
        mesh=_mesh(),
        compiler_params=pltpu.CompilerParams(needs_layout_passes=False),
        scratch_types=(
            pltpu.VMEM((NP,), jnp.float32),
            pltpu.VMEM((NP,), jnp.float32),
            pltpu.VMEM((EPW,), jnp.int32),
            pltpu.VMEM((EPW,), jnp.float32),
            pltpu.VMEM((EPW,), jnp.float32),
        ),
    )
    def lapw_kernel(deg2, rc, ew, lapw, disbuf, tbuf, rcbuf, ewbuf, lwbuf):
        wid = _worker_id()
        pltpu.sync_copy(deg2.at[pl.ds(0, NP)], disbuf)
        pltpu.sync_copy(deg2.at[pl.ds(NP, NP)], tbuf)

        @pl.loop(0, NP // 16, unroll=2)
        def _(i):
            d = disbuf[pl.ds(i * 16, 16)] + tbuf[pl.ds(i * 16, 16)]
            ii = lax.bitcast_convert_type(d, jnp.int32)
            ii = 0x5F3759DF - lax.shift_right_arithmetic(ii, 1)
            y = lax.bitcast_convert_type(ii, jnp.float32)
            for _unused in range(4):
                y = y * (1.5 - 0.5 * d * y * y)
            disbuf[pl.ds(i * 16, 16)] = jnp.where(d > 0.0, y, 0.0)

        base = wid * EPW
        pltpu.sync_copy(rc.at[pl.ds(base, EPW)], rcbuf)
        pltpu.sync_copy(ew.at[pl.ds(base, EPW)], ewbuf)

        @plsc.parallel_loop(0, EPW // 16, unroll=4)
        def _(g):
            rcv = rcbuf[pl.ds(g * 16, 16)]
            rowi = rcv & 0xFFFF
            coli = lax.shift_right_arithmetic(rcv, 16)
            a = plsc.load_gather(disbuf, [rowi])
            b = plsc.load_gather(disbuf, [coli])
            lwbuf[pl.ds(g * 16, 16)] = -(a * ewbuf[pl.ds(g * 16, 16)] * b)

        pltpu.sync_copy(lwbuf, lapw.at[pl.ds(base, EPW)])

    return lapw_kernel


@functools.lru_cache(maxsize=None)
def _build_prop(D):
    FPW = D // NW
    C = 4
    assert FPW % C == 0
    SCANS = FPW // C

    scratch = (
        [pltpu.VMEM((NP,), jnp.float32) for _ in range(2 * C)]
        + [pltpu.VMEM((CHUNK,), jnp.int32) for _ in range(2)]
        + [pltpu.VMEM((CHUNK,), jnp.float32) for _ in range(2)]
        + [pltpu.SemaphoreType.DMA, pltpu.SemaphoreType.DMA]
    )

    @functools.partial(
        pl.kernel,
        out_type=jax.ShapeDtypeStruct((D * NP,), jnp.float32),
        mesh=_mesh(),
        compiler_params=pltpu.CompilerParams(needs_layout_passes=False),
        scratch_types=tuple(scratch),
    )
    def prop(zT, rc, w, outT, *sc):
        zb = sc[0:C]
        ob = sc[C:2 * C]
        rcb = sc[2 * C:2 * C + 2]
        wb = sc[2 * C + 2:2 * C + 4]
        sems = sc[2 * C + 4:2 * C + 6]
        wid = _worker_id()
        z16 = jnp.zeros((16,), jnp.float32)

        def start(chunk_idx, p):
            off = chunk_idx * CHUNK
            pltpu.async_copy(rc.at[pl.ds(off, CHUNK)], rcb[p], sems[p])
            pltpu.async_copy(w.at[pl.ds(off, CHUNK)], wb[p], sems[p])

        def wait(p):
            pltpu.make_async_copy(rc.at[pl.ds(0, CHUNK)], rcb[p], sems[p]).wait()
            pltpu.make_async_copy(w.at[pl.ds(0, CHUNK)], wb[p], sems[p]).wait()

        def process(p):
            @plsc.parallel_loop(0, GPC, unroll=4)
            def _(g):
                rcv = rcb[p][pl.ds(g * 16, 16)]
                rowi = rcv & 0xFFFF
                coli = lax.shift_right_arithmetic(rcv, 16)
                wv = wb[p][pl.ds(g * 16, 16)]
                vals = [plsc.load_gather(zb[q], [rowi]) for q in range(C)]
                for q in range(C):
                    plsc.addupdate_scatter(ob[q], [coli], vals[q] * wv)

        for scan in range(SCANS):
            for q in range(C):
                f = wid * FPW + scan * C + q
                pltpu.sync_copy(zT.at[pl.ds(f * NP, NP)], zb[q])

            @pl.loop(0, NP // 16, unroll=8)
            def _(i):
                for q in range(C):
                    ob[q][pl.ds(i * 16, 16)] = z16

            start(0, 0)

            @pl.loop(0, NCH // 2)
            def _(h):
                ch = h * 2
                start(ch + 1, 1)
                wait(0)
                process(0)

                @pl.when(ch + 2 < NCH)
                def _():
                    start(ch + 2, 0)

                wait(1)
                process(1)

            for q in range(C):
                f = wid * FPW + scan * C + q
                pltpu.sync_copy(ob[q], outT.at[pl.ds(f * NP, NP)])

    return prop


def _prop(D, zT, rc, w):
    return _build_prop(D)(zT.reshape(D * NP), rc, w).reshape(D, NP)


_BN = 2560


def _mm3(w0, w1, w2, z0, z1, z2):
    M, D = w0.shape

    def body(w0r, w1r, w2r, z0r, z1r, z2r, outr):
        acc = jnp.dot(w0r[...], z0r[...], preferred_element_type=jnp.float32)
        acc += jnp.dot(w1r[...], z1r[...], preferred_element_type=jnp.float32)
        acc += jnp.dot(w2r[...], z2r[...], preferred_element_type=jnp.float32)
        outr[...] = acc

    wspec = pl.BlockSpec((M, D), lambda j: (0, 0))
    zspec = pl.BlockSpec((D, _BN), lambda j: (0, j))
    return pl.pallas_call(
        body,
        grid=(NP // _BN,),
        in_specs=[wspec, wspec, wspec, zspec, zspec, zspec],
        out_specs=pl.BlockSpec((M, _BN), lambda j: (0, j)),
        out_shape=jax.ShapeDtypeStruct((M, NP), jnp.float32),
    )(w0, w1, w2, z0, z1, z2)


def _bn_stats(h, g_t, be_t):
    M = h.shape[0]

    def body(hr, gr, br, sr, tr, acc1, acc2):
        j = pl.program_id(0)

        @pl.when(j == 0)
        def _():
            acc1[...] = jnp.zeros_like(acc1)
            acc2[...] = jnp.zeros_like(acc2)

        b = hr[...]
        acc1[...] += b.reshape(M, _BN // 128, 128).sum(axis=1)
        acc2[...] += (b * b).reshape(M, _BN // 128, 128).sum(axis=1)
        ssum = acc1[...].sum(axis=1, keepdims=True)
        sq = acc2[...].sum(axis=1, keepdims=True)
        mean = ssum / N
        var = sq / N - mean * mean
        rstd = lax.rsqrt(var + 1e-5)
        s = gr[...][:, :1] * rstd
        t = br[...][:, :1] - mean * s
        sr[...] = jnp.broadcast_to(s, (M, 128))
        tr[...] = jnp.broadcast_to(t, (M, 128))

    sspec = pl.BlockSpec((M, 128), lambda j: (0, 0))
    return pl.pallas_call(
        body,
        grid=(NP // _BN,),
        in_specs=[pl.BlockSpec((M, _BN), lambda j: (0, j)), sspec, sspec],
        out_specs=[sspec, sspec],
        out_shape=[jax.ShapeDtypeStruct((M, 128), jnp.float32)] * 2,
        scratch_shapes=[pltpu.VMEM((M, 128), jnp.float32)] * 2,
    )(h, g_t, be_t)


def _bn_apply(h, s_t, t_t):
    M = h.shape[0]

    def body(hr, sr, tr, outr):
        j = pl.program_id(0)
        col = j * _BN + lax.broadcasted_iota(jnp.int32, (M, _BN), 1)
        y = hr[...] * sr[...][:, :1] + tr[...][:, :1]
        outr[...] = jnp.where(col < N, y, 0.0)

    sspec = pl.BlockSpec((M, 128), lambda j: (0, 0))
    bspec = pl.BlockSpec((M, _BN), lambda j: (0, j))
    return pl.pallas_call(
        body,
        grid=(NP // _BN,),
        in_specs=[bspec, sspec, sspec],
        out_specs=bspec,
        out_shape=jax.ShapeDtypeStruct((M, NP), jnp.float32),
    )(h, s_t, t_t)


def _final(wlt, h, bl_t):
    def body(wr, hr, br, outr):
        z = jnp.dot(wr[...], hr[...], preferred_element_type=jnp.float32)
        z += br[...][:, :1]
        m = jnp.max(z, axis=0, keepdims=True)
        zc = z - m
        lse = jnp.log(jnp.sum(jnp.exp(zc), axis=0, keepdims=True))
        outr[...] = (zc - lse).T

    return pl.pallas_call(
        body,
        grid=(NP // _BN,),
        in_specs=[
            pl.BlockSpec((OUT_DIM, H2P), lambda j: (0, 0)),
            pl.BlockSpec((H2P, _BN), lambda j: (0, j)),
            pl.BlockSpec((OUT_DIM, 128), lambda j: (0, 0)),
        ],
        out_specs=pl.BlockSpec((_BN, OUT_DIM), lambda j: (j, 0)),
        out_shape=jax.ShapeDtypeStruct((NP, OUT_DIM), jnp.float32),
    )(wlt, h, bl_t)


def kernel(x, edge_index, edge_weight, W1, b1, g1, be1, W2, b2, g2, be2, Wl, bl):
    row = edge_index[0]
    col = edge_index[1]
    rc_raw = jnp.bitwise_or(row, col << 16)
    rc_raw_p = jnp.concatenate([rc_raw, jnp.zeros((EP - E,), jnp.int32)])
    ew_raw_p = jnp.concatenate([edge_weight, jnp.zeros((EP - E,), jnp.float32)])
    deg2 = _build_deg()(rc_raw_p, ew_raw_p)
    lapw_raw = _build_lapw()(deg2, rc_raw_p, ew_raw_p)

    key = ((col & 15) << 4) | ((row - col) & 15)
    perm = jnp.argsort(key, stable=True).reshape(16, E // 16).T.reshape(-1)
    rc_p = jnp.concatenate([rc_raw[perm], jnp.zeros((EP - E,), jnp.int32)])
    lapw = jnp.concatenate([lapw_raw[:E][perm],
                            jnp.zeros((EP - E,), jnp.float32)])

    xT = jnp.pad(x, ((0, NP - N), (0, 0))).T

    T1 = _prop(IN_DIM, xT, rc_p, lapw)
    P2 = _prop(IN_DIM, T1, rc_p, lapw)

    A1 = (W1[0] - W1[2]).T
    B1 = W1[1].T
    C1 = 2.0 * W1[2].T
    h1 = _mm3(A1, B1, C1, xT, T1, P2)

    g1t = jnp.tile(g1[:, None], (1, 128))
    be1t = jnp.tile(be1[:, None], (1, 128))
    s1, t1 = _bn_stats(h1, g1t, be1t)
    h1bn = _bn_apply(h1, s1, t1)

    U1 = _prop(H1, h1bn, rc_p, lapw)
    U2 = _prop(H1, U1, rc_p, lapw)

    pad2 = ((0, H2P - H2), (0, 0))
    A2 = jnp.pad((W2[0] - W2[2]).T, pad2)
    B2 = jnp.pad(W2[1].T, pad2)
    C2 = jnp.pad(2.0 * W2[2].T, pad2)
    h2 = _mm3(A2, B2, C2, h1bn, U1, U2)

    g2t = jnp.tile(jnp.pad(g2, (0, H2P - H2))[:, None], (1, 128))
    be2t = jnp.tile(jnp.pad(be2, (0, H2P - H2))[:, None], (1, 128))
    s2, t2 = _bn_stats(h2, g2t, be2t)

    wlt0 = jnp.pad(Wl.T, ((0, 0), (0, H2P - H2)))
    wlt = wlt0 * s2[:, 0][None, :]
    blf = wlt0 @ t2[:, 0] + bl
    bl_t = jnp.tile(blf[:, None], (1, 128))
    out = _final(wlt, h2, bl_t)
    return out[:N]
